# baseline (speedup 1.0000x reference)
import functools

import jax
import jax.numpy as jnp
from jax import lax
from jax.experimental import pallas as pl
from jax.experimental.pallas import tpu as pltpu
from jax.experimental.pallas import tpu_sc as plsc

_IDX_CHUNK = 128


@functools.lru_cache(maxsize=None)
def _make_gather4(B, D):
    info = plsc.get_sparse_core_info()
    NC, NS = info.num_cores, info.num_subcores
    NW = NC * NS
    assert B % (NW * _IDX_CHUNK) == 0
    bpw = B // NW
    nchunks = bpw // _IDX_CHUNK

    mesh = plsc.VectorSubcoreMesh(core_axis_name="c", subcore_axis_name="s")

    @functools.partial(
        pl.kernel,
        out_type=[jax.ShapeDtypeStruct((B, D), jnp.float32)] * 4,
        mesh=mesh,
        scratch_types=[
            pltpu.VMEM((nchunks, _IDX_CHUNK), jnp.int32),
            pltpu.VMEM((nchunks, _IDX_CHUNK), jnp.int32),
            pltpu.VMEM((bpw, D), jnp.float32),
            pltpu.VMEM((bpw, D), jnp.float32),
            pltpu.VMEM((bpw, D), jnp.float32),
            pltpu.VMEM((bpw, D), jnp.float32),
            pltpu.SemaphoreType.DMA,
        ],
        compiler_params=pltpu.CompilerParams(use_tc_tiling_on_sc=False),
    )
    def gather4(uids, iids, gu_tab, gi_tab, mu_tab, mi_tab,
                out_gu, out_gi, out_mu, out_mi,
                uidx_v, iidx_v, b_gu, b_gi, b_mu, b_mi, sem):
        wid = lax.axis_index("s") * NC + lax.axis_index("c")
        base = wid * bpw
        pltpu.sync_copy(uids.at[wid], uidx_v)
        pltpu.sync_copy(iids.at[wid], iidx_v)
        copies = []
        for j in range(nchunks):
            rows = pl.ds(j * _IDX_CHUNK, _IDX_CHUNK)
            urow = uidx_v.at[j]
            irow = iidx_v.at[j]
            copies.append(pltpu.async_copy(gu_tab.at[urow], b_gu.at[rows], sem))
            copies.append(pltpu.async_copy(gi_tab.at[irow], b_gi.at[rows], sem))
            copies.append(pltpu.async_copy(mu_tab.at[urow], b_mu.at[rows], sem))
            copies.append(pltpu.async_copy(mi_tab.at[irow], b_mi.at[rows], sem))
        for c in copies:
            c.wait()
        out_rows = pl.ds(base, bpw)
        pltpu.sync_copy(b_gu, out_gu.at[out_rows])
        pltpu.sync_copy(b_gi, out_gi.at[out_rows])
        pltpu.sync_copy(b_mu, out_mu.at[out_rows])
        pltpu.sync_copy(b_mi, out_mi.at[out_rows])

    return NW, nchunks, gather4


def _dense_body(gu, gi, mu, mi, w1a, w1b, b1, w2, b2, w3, b3, wog, woh, bo,
                out):
    f32 = jnp.float32
    h = mu[...] @ w1a[...] + mi[...] @ w1b[...] + b1[...]
    h = jnp.maximum(h, 0.0)
    h = jnp.maximum(h @ w2[...] + b2[...], 0.0)
    h = jnp.maximum(h @ w3[...] + b3[...], 0.0)
    g = gu[...] * gi[...]
    logit = g @ wog[...] + h @ woh[...] + bo[...]
    out[...] = (1.0 / (1.0 + jnp.exp(-logit))).astype(f32)


@functools.lru_cache(maxsize=None)
def _make_dense(B, D, H1, H2, H3):
    BLK = 2048
    assert B % BLK == 0
    grid = (B // BLK,)

    def xspec():
        return pl.BlockSpec((BLK, D), lambda i: (i, 0))

    def wspec(shape):
        return pl.BlockSpec(shape, lambda i: (0, 0))

    return pl.pallas_call(
        _dense_body,
        grid=grid,
        in_specs=[
            xspec(), xspec(), xspec(), xspec(),
            wspec((D, H1)), wspec((D, H1)), wspec((1, H1)),
            wspec((H1, H2)), wspec((1, H2)),
            wspec((H2, H3)), wspec((1, H3)),
            wspec((D, 1)), wspec((H3, 1)), wspec((1, 1)),
        ],
        out_specs=pl.BlockSpec((BLK, 1), lambda i: (i, 0)),
        out_shape=jax.ShapeDtypeStruct((B, 1), jnp.float32),
    )


def kernel(user_ids, item_ids, gmf_user_emb, gmf_item_emb, mlp_user_emb,
           mlp_item_emb, W1, b1, W2, b2, W3, b3, Wo, bo):
    B = user_ids.shape[0]
    D = gmf_user_emb.shape[1]
    H1, H2, H3 = W1.shape[1], W2.shape[1], W3.shape[1]

    NW, nchunks, gather4 = _make_gather4(B, D)
    uidx = user_ids.reshape(NW, nchunks, _IDX_CHUNK).astype(jnp.int32)
    iidx = item_ids.reshape(NW, nchunks, _IDX_CHUNK).astype(jnp.int32)
    gu, gi, mu, mi = gather4(uidx, iidx, gmf_user_emb, gmf_item_emb,
                             mlp_user_emb, mlp_item_emb)

    dense = _make_dense(B, D, H1, H2, H3)
    return dense(
        gu, gi, mu, mi,
        W1[:D], W1[D:], b1.reshape(1, H1),
        W2, b2.reshape(1, H2),
        W3, b3.reshape(1, H3),
        Wo[:D], Wo[D:], bo.reshape(1, 1),
    )


# speedup vs baseline: 1.0724x; 1.0724x over previous
import functools

import jax
import jax.numpy as jnp
from jax import lax
from jax.experimental import pallas as pl
from jax.experimental.pallas import tpu as pltpu
from jax.experimental.pallas import tpu_sc as plsc

_IDX_CHUNK = 128
_VBLK = 512


def _detile_body(*refs):
    ins = refs[:4]
    outs = refs[4:]
    for x, o in zip(ins, outs):
        y = jnp.transpose(x[...])
        o[...] = jnp.concatenate(
            [y, jnp.zeros((_VBLK, 128 - y.shape[1]), jnp.float32)], axis=1)


@functools.lru_cache(maxsize=None)
def _make_detile(V, D):
    nblk = (V + _VBLK - 1) // _VBLK
    Vp = nblk * _VBLK

    in_spec = pl.BlockSpec((D, _VBLK), lambda i: (0, i))
    out_spec = pl.BlockSpec((_VBLK, 128), lambda i: (i, 0))
    return Vp, pl.pallas_call(
        _detile_body,
        grid=(nblk,),
        in_specs=[in_spec] * 4,
        out_specs=[out_spec] * 4,
        out_shape=[jax.ShapeDtypeStruct((Vp, 128), jnp.float32)] * 4,
    )


@functools.lru_cache(maxsize=None)
def _make_gather4(B, Vp, D):
    info = plsc.get_sparse_core_info()
    NC, NS = info.num_cores, info.num_subcores
    NW = NC * NS
    assert B % (NW * _IDX_CHUNK) == 0
    bpw = B // NW
    nchunks = bpw // _IDX_CHUNK

    mesh = plsc.VectorSubcoreMesh(core_axis_name="c", subcore_axis_name="s")

    @functools.partial(
        pl.kernel,
        out_type=jax.ShapeDtypeStruct((B, 4 * D), jnp.float32),
        mesh=mesh,
        scratch_types=[
            pltpu.VMEM((nchunks, _IDX_CHUNK), jnp.int32),
            pltpu.VMEM((nchunks, _IDX_CHUNK), jnp.int32),
            pltpu.VMEM((bpw, D), jnp.float32),
            pltpu.VMEM((bpw, D), jnp.float32),
            pltpu.VMEM((bpw, D), jnp.float32),
            pltpu.VMEM((bpw, D), jnp.float32),
            pltpu.SemaphoreType.DMA,
        ],
        compiler_params=pltpu.CompilerParams(use_tc_tiling_on_sc=False),
    )
    def gather4(uids, iids, gu_tab, gi_tab, mu_tab, mi_tab, out_g,
                uidx_v, iidx_v, b_gu, b_gi, b_mu, b_mi, sem):
        wid = lax.axis_index("s") * NC + lax.axis_index("c")
        base = wid * bpw
        pltpu.sync_copy(uids.at[wid], uidx_v)
        pltpu.sync_copy(iids.at[wid], iidx_v)
        copies = []
        for j in range(nchunks):
            rows = pl.ds(j * _IDX_CHUNK, _IDX_CHUNK)
            urow = uidx_v.at[j]
            irow = iidx_v.at[j]
            copies.append(pltpu.async_copy(gu_tab.at[urow], b_gu.at[rows], sem))
            copies.append(pltpu.async_copy(gi_tab.at[irow], b_gi.at[rows], sem))
            copies.append(pltpu.async_copy(mu_tab.at[urow], b_mu.at[rows], sem))
            copies.append(pltpu.async_copy(mi_tab.at[irow], b_mi.at[rows], sem))
        for c in copies:
            c.wait()
        out_rows = pl.ds(base, bpw)
        pltpu.sync_copy(b_gu, out_g.at[out_rows, pl.ds(0 * D, D)])
        pltpu.sync_copy(b_gi, out_g.at[out_rows, pl.ds(1 * D, D)])
        pltpu.sync_copy(b_mu, out_g.at[out_rows, pl.ds(2 * D, D)])
        pltpu.sync_copy(b_mi, out_g.at[out_rows, pl.ds(3 * D, D)])

    return NW, nchunks, gather4


def _dense_body(g_ref, w1a, w1b, b1, w2, b2, w3, b3, wog, woh, bo, out):
    x = g_ref[...]
    D = 32
    gu = x[:, 0 * D:1 * D]
    gi = x[:, 1 * D:2 * D]
    mu = x[:, 2 * D:3 * D]
    mi = x[:, 3 * D:4 * D]
    h = mu @ w1a[...] + mi @ w1b[...] + b1[...]
    h = jnp.maximum(h, 0.0)
    h = jnp.maximum(h @ w2[...] + b2[...], 0.0)
    h = jnp.maximum(h @ w3[...] + b3[...], 0.0)
    g = gu * gi
    logit = g @ wog[...] + h @ woh[...] + bo[...]
    out[...] = 1.0 / (1.0 + jnp.exp(-logit))


@functools.lru_cache(maxsize=None)
def _make_dense(B, D, H1, H2, H3):
    BLK = 2048
    assert B % BLK == 0
    grid = (B // BLK,)

    def wspec(shape):
        return pl.BlockSpec(shape, lambda i: (0, 0))

    return pl.pallas_call(
        _dense_body,
        grid=grid,
        in_specs=[
            pl.BlockSpec((BLK, 4 * D), lambda i: (i, 0)),
            wspec((D, H1)), wspec((D, H1)), wspec((1, H1)),
            wspec((H1, H2)), wspec((1, H2)),
            wspec((H2, H3)), wspec((1, H3)),
            wspec((D, 1)), wspec((H3, 1)), wspec((1, 1)),
        ],
        out_specs=pl.BlockSpec((BLK, 1), lambda i: (i, 0)),
        out_shape=jax.ShapeDtypeStruct((B, 1), jnp.float32),
    )


def kernel(user_ids, item_ids, gmf_user_emb, gmf_item_emb, mlp_user_emb,
           mlp_item_emb, W1, b1, W2, b2, W3, b3, Wo, bo):
    B = user_ids.shape[0]
    V, D = gmf_user_emb.shape
    H1, H2, H3 = W1.shape[1], W2.shape[1], W3.shape[1]

    Vp, detile = _make_detile(V, D)
    pgu, pgi, pmu, pmi = detile(gmf_user_emb.T, gmf_item_emb.T,
                                mlp_user_emb.T, mlp_item_emb.T)
    nrows = Vp * 128 // D
    pgu, pgi, pmu, pmi = (p.reshape(nrows, D) for p in (pgu, pgi, pmu, pmi))

    NW, nchunks, gather4 = _make_gather4(B, Vp, D)
    scale = 128 // D
    uidx = (user_ids.reshape(NW, nchunks, _IDX_CHUNK) * scale).astype(jnp.int32)
    iidx = (item_ids.reshape(NW, nchunks, _IDX_CHUNK) * scale).astype(jnp.int32)
    g = gather4(uidx, iidx, pgu, pgi, pmu, pmi)

    dense = _make_dense(B, D, H1, H2, H3)
    return dense(
        g,
        W1[:D], W1[D:], b1.reshape(1, H1),
        W2, b2.reshape(1, H2),
        W3, b3.reshape(1, H3),
        Wo[:D], Wo[D:], bo.reshape(1, 1),
    )


# speedup vs baseline: 1.4488x; 1.3510x over previous
import functools

import jax
import jax.numpy as jnp
from jax import lax
from jax.experimental import pallas as pl
from jax.experimental.pallas import tpu as pltpu
from jax.experimental.pallas import tpu_sc as plsc

_IDX_CHUNK = 128
_VBLK = 1024


def _detile_body(gu, gi, mu, mi, out):
    out[...] = jnp.concatenate(
        [jnp.transpose(gu[...]), jnp.transpose(gi[...]),
         jnp.transpose(mu[...]), jnp.transpose(mi[...])], axis=1)


@functools.lru_cache(maxsize=None)
def _make_detile(V, D):
    nblk = (V + _VBLK - 1) // _VBLK
    Vp = nblk * _VBLK

    in_spec = pl.BlockSpec((D, _VBLK), lambda i: (0, i))
    return Vp, pl.pallas_call(
        _detile_body,
        grid=(nblk,),
        in_specs=[in_spec] * 4,
        out_specs=pl.BlockSpec((_VBLK, 4 * D), lambda i: (i, 0)),
        out_shape=jax.ShapeDtypeStruct((Vp, 4 * D), jnp.float32),
    )


@functools.lru_cache(maxsize=None)
def _make_gather4(B, D):
    info = plsc.get_sparse_core_info()
    NC, NS = info.num_cores, info.num_subcores
    NW = NC * NS
    assert B % (NW * _IDX_CHUNK) == 0
    bpw = B // NW
    nchunks = bpw // _IDX_CHUNK

    mesh = plsc.VectorSubcoreMesh(core_axis_name="c", subcore_axis_name="s")

    @functools.partial(
        pl.kernel,
        out_type=jax.ShapeDtypeStruct((B, 4 * D), jnp.float32),
        mesh=mesh,
        scratch_types=[
            pltpu.VMEM((nchunks, _IDX_CHUNK), jnp.int32),
            pltpu.VMEM((nchunks, _IDX_CHUNK), jnp.int32),
            pltpu.VMEM((nchunks, _IDX_CHUNK), jnp.int32),
            pltpu.VMEM((nchunks, _IDX_CHUNK), jnp.int32),
            pltpu.VMEM((bpw, D), jnp.float32),
            pltpu.VMEM((bpw, D), jnp.float32),
            pltpu.VMEM((bpw, D), jnp.float32),
            pltpu.VMEM((bpw, D), jnp.float32),
            pltpu.SemaphoreType.DMA,
        ],
        compiler_params=pltpu.CompilerParams(use_tc_tiling_on_sc=False),
    )
    def gather4(i_gu, i_gi, i_mu, i_mi, ptab, out_g,
                v_gu, v_gi, v_mu, v_mi, b_gu, b_gi, b_mu, b_mi, sem):
        wid = lax.axis_index("s") * NC + lax.axis_index("c")
        base = wid * bpw
        pltpu.sync_copy(i_gu.at[wid], v_gu)
        pltpu.sync_copy(i_gi.at[wid], v_gi)
        pltpu.sync_copy(i_mu.at[wid], v_mu)
        pltpu.sync_copy(i_mi.at[wid], v_mi)
        copies = []
        for j in range(nchunks):
            rows = pl.ds(j * _IDX_CHUNK, _IDX_CHUNK)
            copies.append(pltpu.async_copy(ptab.at[v_gu.at[j]], b_gu.at[rows], sem))
            copies.append(pltpu.async_copy(ptab.at[v_gi.at[j]], b_gi.at[rows], sem))
            copies.append(pltpu.async_copy(ptab.at[v_mu.at[j]], b_mu.at[rows], sem))
            copies.append(pltpu.async_copy(ptab.at[v_mi.at[j]], b_mi.at[rows], sem))
        for c in copies:
            c.wait()
        out_rows = pl.ds(base, bpw)
        pltpu.sync_copy(b_gu, out_g.at[out_rows, pl.ds(0 * D, D)])
        pltpu.sync_copy(b_gi, out_g.at[out_rows, pl.ds(1 * D, D)])
        pltpu.sync_copy(b_mu, out_g.at[out_rows, pl.ds(2 * D, D)])
        pltpu.sync_copy(b_mi, out_g.at[out_rows, pl.ds(3 * D, D)])

    return NW, nchunks, gather4


def _dense_body(g_ref, w1a, w1b, b1, w2, b2, w3, b3, wog, woh, bo, out):
    x = g_ref[...]
    D = 32
    gu = x[:, 0 * D:1 * D]
    gi = x[:, 1 * D:2 * D]
    mu = x[:, 2 * D:3 * D]
    mi = x[:, 3 * D:4 * D]
    h = mu @ w1a[...] + mi @ w1b[...] + b1[...]
    h = jnp.maximum(h, 0.0)
    h = jnp.maximum(h @ w2[...] + b2[...], 0.0)
    h = jnp.maximum(h @ w3[...] + b3[...], 0.0)
    logit = (jnp.sum(gu * gi * wog[...], axis=1)
             + jnp.sum(h * woh[...], axis=1) + bo[0, 0])
    out[...] = 1.0 / (1.0 + jnp.exp(-logit))


@functools.lru_cache(maxsize=None)
def _make_dense(B, D, H1, H2, H3):
    BLK = 2048
    assert B % BLK == 0
    grid = (B // BLK,)

    def wspec(shape):
        return pl.BlockSpec(shape, lambda i: (0,) * len(shape))

    return pl.pallas_call(
        _dense_body,
        grid=grid,
        in_specs=[
            pl.BlockSpec((BLK, 4 * D), lambda i: (i, 0)),
            wspec((D, H1)), wspec((D, H1)), wspec((1, H1)),
            wspec((H1, H2)), wspec((1, H2)),
            wspec((H2, H3)), wspec((1, H3)),
            wspec((1, D)), wspec((1, H3)), wspec((1, 1)),
        ],
        out_specs=pl.BlockSpec((BLK,), lambda i: (i,)),
        out_shape=jax.ShapeDtypeStruct((B,), jnp.float32),
    )


def kernel(user_ids, item_ids, gmf_user_emb, gmf_item_emb, mlp_user_emb,
           mlp_item_emb, W1, b1, W2, b2, W3, b3, Wo, bo):
    B = user_ids.shape[0]
    V, D = gmf_user_emb.shape
    H1, H2, H3 = W1.shape[1], W2.shape[1], W3.shape[1]

    Vp, detile = _make_detile(V, D)
    ptab = detile(gmf_user_emb.T, gmf_item_emb.T, mlp_user_emb.T,
                  mlp_item_emb.T)
    ptab = ptab.reshape(Vp * 4, D)

    NW, nchunks, gather4 = _make_gather4(B, D)

    def idx(ids, t):
        return (ids.reshape(NW, nchunks, _IDX_CHUNK) * 4 + t).astype(jnp.int32)

    g = gather4(idx(user_ids, 0), idx(item_ids, 1),
                idx(user_ids, 2), idx(item_ids, 3), ptab)

    dense = _make_dense(B, D, H1, H2, H3)
    out = dense(
        g,
        W1[:D], W1[D:], b1.reshape(1, H1),
        W2, b2.reshape(1, H2),
        W3, b3.reshape(1, H3),
        Wo[:D].reshape(1, D), Wo[D:].reshape(1, H3), bo.reshape(1, 1),
    )
    return out.reshape(B, 1)


# speedup vs baseline: 1.8577x; 1.2822x over previous
import functools

import jax
import jax.numpy as jnp
from jax import lax
from jax.experimental import pallas as pl
from jax.experimental.pallas import tpu as pltpu
from jax.experimental.pallas import tpu_sc as plsc

_IDX_CHUNK = 128
_VBLK = 1024


def _detile_body(gu, gi, mu, mi, ident, out):
    x = jnp.concatenate([gu[...], gi[...], mu[...], mi[...]], axis=0)
    out[...] = jax.lax.dot_general(
        x, ident[...], (((0,), (0,)), ((), ())),
        preferred_element_type=jnp.float32)


@functools.lru_cache(maxsize=None)
def _make_detile(V, D):
    nblk = (V + _VBLK - 1) // _VBLK
    Vp = nblk * _VBLK

    in_spec = pl.BlockSpec((D, _VBLK), lambda i: (0, i))
    return Vp, pl.pallas_call(
        _detile_body,
        grid=(nblk,),
        in_specs=[in_spec] * 4 + [
            pl.BlockSpec((4 * D, 4 * D), lambda i: (0, 0))],
        out_specs=pl.BlockSpec((_VBLK, 4 * D), lambda i: (i, 0)),
        out_shape=jax.ShapeDtypeStruct((Vp, 4 * D), jnp.float32),
        compiler_params=pltpu.CompilerParams(
            fuse_transposed_lhs_in_matmul=True),
    )


@functools.lru_cache(maxsize=None)
def _make_gather4(B, D):
    info = plsc.get_sparse_core_info()
    NC, NS = info.num_cores, info.num_subcores
    NW = NC * NS
    assert B % (NW * _IDX_CHUNK) == 0
    bpw = B // NW
    nchunks = bpw // _IDX_CHUNK

    mesh = plsc.VectorSubcoreMesh(core_axis_name="c", subcore_axis_name="s")

    @functools.partial(
        pl.kernel,
        out_type=jax.ShapeDtypeStruct((B, 4 * D), jnp.float32),
        mesh=mesh,
        scratch_types=[
            pltpu.VMEM((nchunks, _IDX_CHUNK), jnp.int32),
            pltpu.VMEM((nchunks, _IDX_CHUNK), jnp.int32),
            pltpu.VMEM((nchunks, _IDX_CHUNK), jnp.int32),
            pltpu.VMEM((nchunks, _IDX_CHUNK), jnp.int32),
            pltpu.VMEM((bpw, D), jnp.float32),
            pltpu.VMEM((bpw, D), jnp.float32),
            pltpu.VMEM((bpw, D), jnp.float32),
            pltpu.VMEM((bpw, D), jnp.float32),
            pltpu.SemaphoreType.DMA,
        ],
        compiler_params=pltpu.CompilerParams(use_tc_tiling_on_sc=False),
    )
    def gather4(i_gu, i_gi, i_mu, i_mi, ptab, out_g,
                v_gu, v_gi, v_mu, v_mi, b_gu, b_gi, b_mu, b_mi, sem):
        wid = lax.axis_index("s") * NC + lax.axis_index("c")
        base = wid * bpw
        pltpu.sync_copy(i_gu.at[wid], v_gu)
        pltpu.sync_copy(i_gi.at[wid], v_gi)
        pltpu.sync_copy(i_mu.at[wid], v_mu)
        pltpu.sync_copy(i_mi.at[wid], v_mi)
        copies = []
        for j in range(nchunks):
            rows = pl.ds(j * _IDX_CHUNK, _IDX_CHUNK)
            copies.append(pltpu.async_copy(ptab.at[v_gu.at[j]], b_gu.at[rows], sem))
            copies.append(pltpu.async_copy(ptab.at[v_gi.at[j]], b_gi.at[rows], sem))
            copies.append(pltpu.async_copy(ptab.at[v_mu.at[j]], b_mu.at[rows], sem))
            copies.append(pltpu.async_copy(ptab.at[v_mi.at[j]], b_mi.at[rows], sem))
        for c in copies:
            c.wait()
        out_rows = pl.ds(base, bpw)
        pltpu.sync_copy(b_gu, out_g.at[out_rows, pl.ds(0 * D, D)])
        pltpu.sync_copy(b_gi, out_g.at[out_rows, pl.ds(1 * D, D)])
        pltpu.sync_copy(b_mu, out_g.at[out_rows, pl.ds(2 * D, D)])
        pltpu.sync_copy(b_mi, out_g.at[out_rows, pl.ds(3 * D, D)])

    return NW, nchunks, gather4


def _dense_body(g_ref, w1a, w1b, b1, w2, b2, w3, b3, wog_b, woh_b, bo,
                tile_eye, sel, out):
    x = g_ref[...]
    D = 32
    gu = x[:, 0 * D:1 * D]
    gi = x[:, 1 * D:2 * D]
    mu = x[:, 2 * D:3 * D]
    mi = x[:, 3 * D:4 * D]
    h = mu @ w1a[...] + mi @ w1b[...] + b1[...]
    h = jnp.maximum(h, 0.0)
    h = jnp.maximum(h @ w2[...] + b2[...], 0.0)
    h = jnp.maximum(h @ w3[...] + b3[...], 0.0)
    g = gu * gi
    logit_b = g @ wog_b[...] + h @ woh_b[...]
    z = logit_b * tile_eye[...]
    y = jax.lax.dot_general(sel[...], z, (((0,), (0,)), ((), ())),
                            preferred_element_type=jnp.float32)
    y = y + bo[0, 0]
    out[...] = (1.0 / (1.0 + jnp.exp(-y))).reshape(-1)


@functools.lru_cache(maxsize=None)
def _make_dense(B, D, H1, H2, H3):
    BLK = 2048
    assert B % BLK == 0
    grid = (B // BLK,)

    def wspec(shape):
        return pl.BlockSpec(shape, lambda i: (0,) * len(shape))

    return pl.pallas_call(
        _dense_body,
        grid=grid,
        in_specs=[
            pl.BlockSpec((BLK, 4 * D), lambda i: (i, 0)),
            wspec((D, H1)), wspec((D, H1)), wspec((1, H1)),
            wspec((H1, H2)), wspec((1, H2)),
            wspec((H2, H3)), wspec((1, H3)),
            wspec((D, 4 * D)), wspec((H3, 4 * D)), wspec((1, 1)),
            wspec((BLK, 4 * D)), wspec((BLK, BLK // (4 * D))),
        ],
        out_specs=pl.BlockSpec((BLK,), lambda i: (i,)),
        out_shape=jax.ShapeDtypeStruct((B,), jnp.float32),
    )


def kernel(user_ids, item_ids, gmf_user_emb, gmf_item_emb, mlp_user_emb,
           mlp_item_emb, W1, b1, W2, b2, W3, b3, Wo, bo):
    B = user_ids.shape[0]
    V, D = gmf_user_emb.shape
    H1, H2, H3 = W1.shape[1], W2.shape[1], W3.shape[1]

    Vp, detile = _make_detile(V, D)
    ident = jnp.eye(4 * D, dtype=jnp.float32)
    ptab = detile(gmf_user_emb.T, gmf_item_emb.T, mlp_user_emb.T,
                  mlp_item_emb.T, ident)
    ptab = ptab.reshape(Vp * 4, D)

    NW, nchunks, gather4 = _make_gather4(B, D)

    def idx(ids, t):
        return (ids.reshape(NW, nchunks, _IDX_CHUNK) * 4 + t).astype(jnp.int32)

    g = gather4(idx(user_ids, 0), idx(item_ids, 1),
                idx(user_ids, 2), idx(item_ids, 3), ptab)

    dense = _make_dense(B, D, H1, H2, H3)
    BLK = 2048
    ngrp = BLK // (4 * D)
    wog_b = jnp.broadcast_to(Wo[:D], (D, 4 * D))
    woh_b = jnp.broadcast_to(Wo[D:], (H3, 4 * D))
    tile_eye = jnp.tile(jnp.eye(4 * D, dtype=jnp.float32), (ngrp, 1))
    sel = jnp.repeat(jnp.eye(ngrp, dtype=jnp.float32), 4 * D, axis=0)
    out = dense(
        g,
        W1[:D], W1[D:], b1.reshape(1, H1),
        W2, b2.reshape(1, H2),
        W3, b3.reshape(1, H3),
        wog_b, woh_b, bo.reshape(1, 1),
        tile_eye, sel,
    )
    return out.reshape(B, 1)


# speedup vs baseline: 2.4041x; 1.2942x over previous
import functools

import jax
import jax.numpy as jnp
from jax import lax
from jax.experimental import pallas as pl
from jax.experimental.pallas import tpu as pltpu
from jax.experimental.pallas import tpu_sc as plsc

_IDX_CHUNK = 128
_VBLK = 2048


def _detile_body(gu, gi, mu, mi, ident, out):
    x = jnp.concatenate([gu[...], gi[...], mu[...], mi[...]], axis=0)
    out[...] = jax.lax.dot_general(
        x, ident[...], (((0,), (0,)), ((), ())),
        preferred_element_type=jnp.float32)


@functools.lru_cache(maxsize=None)
def _make_detile(V, D):
    nblk = (V + _VBLK - 1) // _VBLK
    Vp = nblk * _VBLK

    in_spec = pl.BlockSpec((D, _VBLK), lambda i: (0, i))
    return Vp, pl.pallas_call(
        _detile_body,
        grid=(nblk,),
        in_specs=[in_spec] * 4 + [
            pl.BlockSpec((4 * D, 4 * D), lambda i: (0, 0))],
        out_specs=pl.BlockSpec((_VBLK, 4 * D), lambda i: (i, 0)),
        out_shape=jax.ShapeDtypeStruct((Vp, 4 * D), jnp.float32),
        compiler_params=pltpu.CompilerParams(
            fuse_transposed_lhs_in_matmul=True),
    )


@functools.lru_cache(maxsize=None)
def _make_gather4(B, D):
    info = plsc.get_sparse_core_info()
    NC, NS = info.num_cores, info.num_subcores
    NW = NC * NS
    assert B % (NW * _IDX_CHUNK) == 0
    bpw = B // NW
    nchunks = bpw // _IDX_CHUNK

    mesh = plsc.VectorSubcoreMesh(core_axis_name="c", subcore_axis_name="s")

    @functools.partial(
        pl.kernel,
        out_type=jax.ShapeDtypeStruct((B, 4 * D), jnp.float32),
        mesh=mesh,
        scratch_types=[
            pltpu.VMEM((nchunks, _IDX_CHUNK), jnp.int32),
            pltpu.VMEM((nchunks, _IDX_CHUNK), jnp.int32),
            pltpu.VMEM((nchunks, _IDX_CHUNK), jnp.int32),
            pltpu.VMEM((nchunks, _IDX_CHUNK), jnp.int32),
            pltpu.VMEM((bpw, D), jnp.float32),
            pltpu.VMEM((bpw, D), jnp.float32),
            pltpu.VMEM((bpw, D), jnp.float32),
            pltpu.VMEM((bpw, D), jnp.float32),
            pltpu.SemaphoreType.DMA,
        ],
        compiler_params=pltpu.CompilerParams(use_tc_tiling_on_sc=False),
    )
    def gather4(i_gu, i_gi, i_mu, i_mi, ptab, out_g,
                v_gu, v_gi, v_mu, v_mi, b_gu, b_gi, b_mu, b_mi, sem):
        wid = lax.axis_index("s") * NC + lax.axis_index("c")
        base = wid * bpw
        pltpu.sync_copy(i_gu.at[wid], v_gu)
        pltpu.sync_copy(i_gi.at[wid], v_gi)
        pltpu.sync_copy(i_mu.at[wid], v_mu)
        pltpu.sync_copy(i_mi.at[wid], v_mi)
        copies = []
        for j in range(nchunks):
            rows = pl.ds(j * _IDX_CHUNK, _IDX_CHUNK)
            copies.append(pltpu.async_copy(ptab.at[v_gu.at[j]], b_gu.at[rows], sem))
            copies.append(pltpu.async_copy(ptab.at[v_gi.at[j]], b_gi.at[rows], sem))
            copies.append(pltpu.async_copy(ptab.at[v_mu.at[j]], b_mu.at[rows], sem))
            copies.append(pltpu.async_copy(ptab.at[v_mi.at[j]], b_mi.at[rows], sem))
        for c in copies:
            c.wait()
        out_rows = pl.ds(base, bpw)
        pltpu.sync_copy(b_gu, out_g.at[out_rows, pl.ds(0 * D, D)])
        pltpu.sync_copy(b_gi, out_g.at[out_rows, pl.ds(1 * D, D)])
        pltpu.sync_copy(b_mu, out_g.at[out_rows, pl.ds(2 * D, D)])
        pltpu.sync_copy(b_mi, out_g.at[out_rows, pl.ds(3 * D, D)])

    return NW, nchunks, gather4


def _dense_body(g_ref, w1a, w1b, b1, w2, b2, w3, b3, wog, woh, bo, out):
    x = g_ref[...]
    D = 32
    BLK = x.shape[0]
    L = 4 * D
    gu = x[:, 0 * D:1 * D]
    gi = x[:, 1 * D:2 * D]
    mu = x[:, 2 * D:3 * D]
    mi = x[:, 3 * D:4 * D]
    h = mu @ w1a[...] + mi @ w1b[...] + b1[...]
    h = jnp.maximum(h, 0.0)
    h = jnp.maximum(h @ w2[...] + b2[...], 0.0)
    h = jnp.maximum(h @ w3[...] + b3[...], 0.0)
    g = gu * gi
    wog_b = jnp.broadcast_to(wog[...], (D, L))
    woh_b = jnp.broadcast_to(woh[...], (H := h.shape[1], L))
    logit_b = g @ wog_b + h @ woh_b
    rows = jax.lax.broadcasted_iota(jnp.int32, (BLK, L), 0)
    cols = jax.lax.broadcasted_iota(jnp.int32, (BLK, L), 1)
    z = jnp.where(rows % L == cols, logit_b, 0.0)
    ngrp = BLK // L
    sel_rows = jax.lax.broadcasted_iota(jnp.int32, (BLK, ngrp), 0)
    sel_cols = jax.lax.broadcasted_iota(jnp.int32, (BLK, ngrp), 1)
    sel = jnp.where(sel_rows // L == sel_cols, 1.0, 0.0)
    y = jax.lax.dot_general(sel, z, (((0,), (0,)), ((), ())),
                            preferred_element_type=jnp.float32)
    y = y + bo[0, 0]
    out[...] = (1.0 / (1.0 + jnp.exp(-y))).reshape(-1)


@functools.lru_cache(maxsize=None)
def _make_dense(B, D, H1, H2, H3):
    BLK = 4096
    assert B % BLK == 0
    grid = (B // BLK,)

    def wspec(shape):
        return pl.BlockSpec(shape, lambda i: (0,) * len(shape))

    return pl.pallas_call(
        _dense_body,
        grid=grid,
        in_specs=[
            pl.BlockSpec((BLK, 4 * D), lambda i: (i, 0)),
            wspec((D, H1)), wspec((D, H1)), wspec((1, H1)),
            wspec((H1, H2)), wspec((1, H2)),
            wspec((H2, H3)), wspec((1, H3)),
            wspec((D, 1)), wspec((H3, 1)), wspec((1, 1)),
        ],
        out_specs=pl.BlockSpec((BLK,), lambda i: (i,)),
        out_shape=jax.ShapeDtypeStruct((B,), jnp.float32),
    )


def kernel(user_ids, item_ids, gmf_user_emb, gmf_item_emb, mlp_user_emb,
           mlp_item_emb, W1, b1, W2, b2, W3, b3, Wo, bo):
    B = user_ids.shape[0]
    V, D = gmf_user_emb.shape
    H1, H2, H3 = W1.shape[1], W2.shape[1], W3.shape[1]

    Vp, detile = _make_detile(V, D)
    ident = jnp.eye(4 * D, dtype=jnp.float32)
    ptab = detile(gmf_user_emb.T, gmf_item_emb.T, mlp_user_emb.T,
                  mlp_item_emb.T, ident)
    ptab = ptab.reshape(Vp * 4, D)

    NW, nchunks, gather4 = _make_gather4(B, D)

    def idx(ids, t):
        return (ids.reshape(NW, nchunks, _IDX_CHUNK) * 4 + t).astype(jnp.int32)

    g = gather4(idx(user_ids, 0), idx(item_ids, 1),
                idx(user_ids, 2), idx(item_ids, 3), ptab)

    dense = _make_dense(B, D, H1, H2, H3)
    out = dense(
        g,
        W1[:D], W1[D:], b1.reshape(1, H1),
        W2, b2.reshape(1, H2),
        W3, b3.reshape(1, H3),
        Wo[:D], Wo[D:], bo.reshape(1, 1),
    )
    return out.reshape(B, 1)


# speedup vs baseline: 3.0429x; 1.2657x over previous
import functools

import jax
import jax.numpy as jnp
from jax import lax
from jax.experimental import pallas as pl
from jax.experimental.pallas import tpu as pltpu
from jax.experimental.pallas import tpu_sc as plsc

_IDX_CHUNK = 128
_VBLK = 4096


def _detile_body(gu, gi, mu, mi, ident, out):
    x = jnp.concatenate([gu[...], gi[...], mu[...], mi[...]], axis=0)
    out[...] = jax.lax.dot_general(
        x, ident[...], (((0,), (0,)), ((), ())),
        preferred_element_type=jnp.float32)


@functools.lru_cache(maxsize=None)
def _make_detile(V, D):
    nblk = (V + _VBLK - 1) // _VBLK
    Vp = nblk * _VBLK

    in_spec = pl.BlockSpec((D, _VBLK), lambda i: (0, i))
    return Vp, pl.pallas_call(
        _detile_body,
        grid=(nblk,),
        in_specs=[in_spec] * 4 + [
            pl.BlockSpec((4 * D, 4 * D), lambda i: (0, 0))],
        out_specs=pl.BlockSpec((_VBLK, 4 * D), lambda i: (i, 0)),
        out_shape=jax.ShapeDtypeStruct((Vp, 4 * D), jnp.float32),
        compiler_params=pltpu.CompilerParams(
            fuse_transposed_lhs_in_matmul=True),
    )


@functools.lru_cache(maxsize=None)
def _make_gather4(B, D):
    info = plsc.get_sparse_core_info()
    NC, NS = info.num_cores, info.num_subcores
    NW = NC * NS
    assert B % (NW * _IDX_CHUNK) == 0
    bpw = B // NW
    nchunks = bpw // _IDX_CHUNK

    mesh = plsc.VectorSubcoreMesh(core_axis_name="c", subcore_axis_name="s")

    L = 16

    @functools.partial(
        pl.kernel,
        out_type=jax.ShapeDtypeStruct((B, 4 * D), jnp.float32),
        mesh=mesh,
        scratch_types=[
            pltpu.VMEM((bpw,), jnp.int32),
            pltpu.VMEM((bpw,), jnp.int32),
            pltpu.VMEM((nchunks, _IDX_CHUNK), jnp.int32),
            pltpu.VMEM((nchunks, _IDX_CHUNK), jnp.int32),
            pltpu.VMEM((nchunks, _IDX_CHUNK), jnp.int32),
            pltpu.VMEM((nchunks, _IDX_CHUNK), jnp.int32),
            pltpu.VMEM((bpw, D), jnp.float32),
            pltpu.VMEM((bpw, D), jnp.float32),
            pltpu.VMEM((bpw, D), jnp.float32),
            pltpu.VMEM((bpw, D), jnp.float32),
            pltpu.SemaphoreType.DMA,
            pltpu.SemaphoreType.DMA,
            pltpu.SemaphoreType.DMA,
            pltpu.SemaphoreType.DMA,
            pltpu.SemaphoreType.DMA,
        ],
        compiler_params=pltpu.CompilerParams(use_tc_tiling_on_sc=False),
    )
    def gather4(uids, iids, ptab, out_g,
                u_raw, i_raw, v_gu, v_gi, v_mu, v_mi,
                b_gu, b_gi, b_mu, b_mi,
                s_gu, s_gi, s_mu, s_mi, s_wr):
        wid = lax.axis_index("s") * NC + lax.axis_index("c")
        base = wid * bpw
        pltpu.sync_copy(uids.at[wid], u_raw)
        pltpu.sync_copy(iids.at[wid], i_raw)
        for j in range(bpw // L):
            r, c = j // (_IDX_CHUNK // L), j % (_IDX_CHUNK // L)
            sl = pl.ds(c * L, L)
            mu4 = u_raw[pl.ds(j * L, L)] * 4
            mi4 = i_raw[pl.ds(j * L, L)] * 4
            v_gu[r, sl] = mu4
            v_gi[r, sl] = mi4 + 1
            v_mu[r, sl] = mu4 + 2
            v_mi[r, sl] = mi4 + 3
        copies = []
        for j in range(nchunks):
            rows = pl.ds(j * _IDX_CHUNK, _IDX_CHUNK)
            copies.append(pltpu.async_copy(ptab.at[v_gu.at[j]], b_gu.at[rows], s_gu))
            copies.append(pltpu.async_copy(ptab.at[v_gi.at[j]], b_gi.at[rows], s_gi))
            copies.append(pltpu.async_copy(ptab.at[v_mu.at[j]], b_mu.at[rows], s_mu))
            copies.append(pltpu.async_copy(ptab.at[v_mi.at[j]], b_mi.at[rows], s_mi))
        out_rows = pl.ds(base, bpw)
        writes = []
        for t, (buf, sem) in enumerate(
                [(b_gu, s_gu), (b_gi, s_gi), (b_mu, s_mu), (b_mi, s_mi)]):
            for c in copies[t::4]:
                c.wait()
            writes.append(pltpu.async_copy(
                buf, out_g.at[out_rows, pl.ds(t * D, D)], s_wr))
        for w in writes:
            w.wait()

    return NW, nchunks, gather4


def _dense_body(g_ref, w1a, w1b, b1, w2, b2, w3, b3, wog, woh, bo, out):
    x = g_ref[...]
    D = 32
    BLK = x.shape[0]
    L = 4 * D
    gu = x[:, 0 * D:1 * D]
    gi = x[:, 1 * D:2 * D]
    mu = x[:, 2 * D:3 * D]
    mi = x[:, 3 * D:4 * D]
    h = mu @ w1a[...] + mi @ w1b[...] + b1[...]
    h = jnp.maximum(h, 0.0)
    h = jnp.maximum(h @ w2[...] + b2[...], 0.0)
    h = jnp.maximum(h @ w3[...] + b3[...], 0.0)
    g = gu * gi
    wog_b = jnp.broadcast_to(wog[...], (D, L))
    woh_b = jnp.broadcast_to(woh[...], (H := h.shape[1], L))
    logit_b = g @ wog_b + h @ woh_b
    rows = jax.lax.broadcasted_iota(jnp.int32, (BLK, L), 0)
    cols = jax.lax.broadcasted_iota(jnp.int32, (BLK, L), 1)
    z = jnp.where(rows % L == cols, logit_b, 0.0)
    ngrp = BLK // L
    sel_rows = jax.lax.broadcasted_iota(jnp.int32, (BLK, ngrp), 0)
    sel_cols = jax.lax.broadcasted_iota(jnp.int32, (BLK, ngrp), 1)
    sel = jnp.where(sel_rows // L == sel_cols, 1.0, 0.0)
    y = jax.lax.dot_general(sel, z, (((0,), (0,)), ((), ())),
                            preferred_element_type=jnp.float32)
    y = y + bo[0, 0]
    out[...] = (1.0 / (1.0 + jnp.exp(-y))).reshape(-1)


@functools.lru_cache(maxsize=None)
def _make_dense(B, D, H1, H2, H3):
    BLK = 8192
    assert B % BLK == 0
    grid = (B // BLK,)

    def wspec(shape):
        return pl.BlockSpec(shape, lambda i: (0,) * len(shape))

    return pl.pallas_call(
        _dense_body,
        grid=grid,
        in_specs=[
            pl.BlockSpec((BLK, 4 * D), lambda i: (i, 0)),
            wspec((D, H1)), wspec((D, H1)), wspec((1, H1)),
            wspec((H1, H2)), wspec((1, H2)),
            wspec((H2, H3)), wspec((1, H3)),
            wspec((D, 1)), wspec((H3, 1)), wspec((1, 1)),
        ],
        out_specs=pl.BlockSpec((BLK,), lambda i: (i,)),
        out_shape=jax.ShapeDtypeStruct((B,), jnp.float32),
    )


def kernel(user_ids, item_ids, gmf_user_emb, gmf_item_emb, mlp_user_emb,
           mlp_item_emb, W1, b1, W2, b2, W3, b3, Wo, bo):
    B = user_ids.shape[0]
    V, D = gmf_user_emb.shape
    H1, H2, H3 = W1.shape[1], W2.shape[1], W3.shape[1]

    Vp, detile = _make_detile(V, D)
    ident = jnp.eye(4 * D, dtype=jnp.float32)
    ptab = detile(gmf_user_emb.T, gmf_item_emb.T, mlp_user_emb.T,
                  mlp_item_emb.T, ident)
    ptab = ptab.reshape(Vp * 4, D)

    NW, nchunks, gather4 = _make_gather4(B, D)
    bpw = B // NW
    g = gather4(user_ids.reshape(NW, bpw).astype(jnp.int32),
                item_ids.reshape(NW, bpw).astype(jnp.int32), ptab)

    dense = _make_dense(B, D, H1, H2, H3)
    out = dense(
        g,
        W1[:D], W1[D:], b1.reshape(1, H1),
        W2, b2.reshape(1, H2),
        W3, b3.reshape(1, H3),
        Wo[:D], Wo[D:], bo.reshape(1, 1),
    )
    return out.reshape(B, 1)


# speedup vs baseline: 3.3717x; 1.1081x over previous
import functools

import jax
import jax.numpy as jnp
from jax import lax
from jax.experimental import pallas as pl
from jax.experimental.pallas import tpu as pltpu
from jax.experimental.pallas import tpu_sc as plsc

_IDX_CHUNK = 128
_VBLK = 8192


def _detile_body(gu, gi, mu, mi, ident, out):
    x = jnp.concatenate([gu[...], gi[...], mu[...], mi[...]], axis=0)
    out[...] = jax.lax.dot_general(
        x, ident[...], (((0,), (0,)), ((), ())),
        preferred_element_type=jnp.float32)


@functools.lru_cache(maxsize=None)
def _make_detile(V, D):
    nblk = (V + _VBLK - 1) // _VBLK
    Vp = nblk * _VBLK

    in_spec = pl.BlockSpec((D, _VBLK), lambda i: (0, i))
    return Vp, pl.pallas_call(
        _detile_body,
        grid=(nblk,),
        in_specs=[in_spec] * 4 + [
            pl.BlockSpec((4 * D, 4 * D), lambda i: (0, 0))],
        out_specs=pl.BlockSpec((_VBLK, 4 * D), lambda i: (i, 0)),
        out_shape=jax.ShapeDtypeStruct((Vp, 4 * D), jnp.float32),
        compiler_params=pltpu.CompilerParams(
            fuse_transposed_lhs_in_matmul=True),
    )


@functools.lru_cache(maxsize=None)
def _make_gather4(B, D):
    info = plsc.get_sparse_core_info()
    NC, NS = info.num_cores, info.num_subcores
    NW = NC * NS
    assert B % (NW * _IDX_CHUNK) == 0
    bpw = B // NW
    nchunks = bpw // _IDX_CHUNK

    mesh = plsc.VectorSubcoreMesh(core_axis_name="c", subcore_axis_name="s")

    L = 16

    @functools.partial(
        pl.kernel,
        out_type=jax.ShapeDtypeStruct((B, 4 * D), jnp.float32),
        mesh=mesh,
        scratch_types=[
            pltpu.VMEM((bpw,), jnp.int32),
            pltpu.VMEM((bpw,), jnp.int32),
            pltpu.VMEM((nchunks, _IDX_CHUNK), jnp.int32),
            pltpu.VMEM((nchunks, _IDX_CHUNK), jnp.int32),
            pltpu.VMEM((nchunks, _IDX_CHUNK), jnp.int32),
            pltpu.VMEM((nchunks, _IDX_CHUNK), jnp.int32),
            pltpu.VMEM((bpw, D), jnp.float32),
            pltpu.VMEM((bpw, D), jnp.float32),
            pltpu.VMEM((bpw, D), jnp.float32),
            pltpu.VMEM((bpw, D), jnp.float32),
            pltpu.SemaphoreType.DMA,
            pltpu.SemaphoreType.DMA,
            pltpu.SemaphoreType.DMA,
            pltpu.SemaphoreType.DMA,
            pltpu.SemaphoreType.DMA,
        ],
        compiler_params=pltpu.CompilerParams(use_tc_tiling_on_sc=False),
    )
    def gather4(uids, iids, ptab, out_g,
                u_raw, i_raw, v_gu, v_gi, v_mu, v_mi,
                b_gu, b_gi, b_mu, b_mi,
                s_gu, s_gi, s_mu, s_mi, s_wr):
        wid = lax.axis_index("s") * NC + lax.axis_index("c")
        base = wid * bpw
        pltpu.sync_copy(uids.at[wid], u_raw)
        pltpu.sync_copy(iids.at[wid], i_raw)
        for j in range(bpw // L):
            r, c = j // (_IDX_CHUNK // L), j % (_IDX_CHUNK // L)
            sl = pl.ds(c * L, L)
            mu4 = u_raw[pl.ds(j * L, L)] * 4
            mi4 = i_raw[pl.ds(j * L, L)] * 4
            v_gu[r, sl] = mu4
            v_gi[r, sl] = mi4 + 1
            v_mu[r, sl] = mu4 + 2
            v_mi[r, sl] = mi4 + 3
        copies = []
        for j in range(nchunks):
            rows = pl.ds(j * _IDX_CHUNK, _IDX_CHUNK)
            copies.append(pltpu.async_copy(ptab.at[v_gu.at[j]], b_gu.at[rows], s_gu))
            copies.append(pltpu.async_copy(ptab.at[v_gi.at[j]], b_gi.at[rows], s_gi))
            copies.append(pltpu.async_copy(ptab.at[v_mu.at[j]], b_mu.at[rows], s_mu))
            copies.append(pltpu.async_copy(ptab.at[v_mi.at[j]], b_mi.at[rows], s_mi))
        out_rows = pl.ds(base, bpw)
        writes = []
        for t, (buf, sem) in enumerate(
                [(b_gu, s_gu), (b_gi, s_gi), (b_mu, s_mu), (b_mi, s_mi)]):
            for c in copies[t::4]:
                c.wait()
            writes.append(pltpu.async_copy(
                buf, out_g.at[out_rows, pl.ds(t * D, D)], s_wr))
        for w in writes:
            w.wait()

    return NW, nchunks, gather4


def _dense_body(g_ref, w1f, b1, w2, b2, w3, b3, wog, woh, bo, out):
    x = g_ref[...]
    D = 32
    BLK = x.shape[0]
    L = 4 * D
    gu = x[:, 0 * D:1 * D]
    gi = x[:, 1 * D:2 * D]
    h = x @ w1f[...] + b1[...]
    h = jnp.maximum(h, 0.0)
    h = jnp.maximum(h @ w2[...] + b2[...], 0.0)
    h = jnp.maximum(h @ w3[...] + b3[...], 0.0)
    g = gu * gi
    wog_b = jnp.broadcast_to(wog[...], (D, L))
    woh_b = jnp.broadcast_to(woh[...], (H := h.shape[1], L))
    logit_b = g @ wog_b + h @ woh_b
    rows = jax.lax.broadcasted_iota(jnp.int32, (BLK, L), 0)
    cols = jax.lax.broadcasted_iota(jnp.int32, (BLK, L), 1)
    z = jnp.where(rows % L == cols, logit_b, 0.0)
    ngrp = BLK // L
    sel_rows = jax.lax.broadcasted_iota(jnp.int32, (BLK, ngrp), 0)
    sel_cols = jax.lax.broadcasted_iota(jnp.int32, (BLK, ngrp), 1)
    sel = jnp.where(sel_rows // L == sel_cols, 1.0, 0.0)
    y = jax.lax.dot_general(sel, z, (((0,), (0,)), ((), ())),
                            preferred_element_type=jnp.float32)
    y = y + bo[0, 0]
    out[...] = (1.0 / (1.0 + jnp.exp(-y))).reshape(-1)


@functools.lru_cache(maxsize=None)
def _make_dense(B, D, H1, H2, H3):
    BLK = 8192
    assert B % BLK == 0
    grid = (B // BLK,)

    def wspec(shape):
        return pl.BlockSpec(shape, lambda i: (0,) * len(shape))

    return pl.pallas_call(
        _dense_body,
        grid=grid,
        in_specs=[
            pl.BlockSpec((BLK, 4 * D), lambda i: (i, 0)),
            wspec((4 * D, H1)), wspec((1, H1)),
            wspec((H1, H2)), wspec((1, H2)),
            wspec((H2, H3)), wspec((1, H3)),
            wspec((D, 1)), wspec((H3, 1)), wspec((1, 1)),
        ],
        out_specs=pl.BlockSpec((BLK,), lambda i: (i,)),
        out_shape=jax.ShapeDtypeStruct((B,), jnp.float32),
    )


def kernel(user_ids, item_ids, gmf_user_emb, gmf_item_emb, mlp_user_emb,
           mlp_item_emb, W1, b1, W2, b2, W3, b3, Wo, bo):
    B = user_ids.shape[0]
    V, D = gmf_user_emb.shape
    H1, H2, H3 = W1.shape[1], W2.shape[1], W3.shape[1]

    Vp, detile = _make_detile(V, D)
    ident = jnp.eye(4 * D, dtype=jnp.float32)
    ptab = detile(gmf_user_emb.T, gmf_item_emb.T, mlp_user_emb.T,
                  mlp_item_emb.T, ident)
    ptab = ptab.reshape(Vp * 4, D)

    NW, nchunks, gather4 = _make_gather4(B, D)
    bpw = B // NW
    g = gather4(user_ids.reshape(NW, bpw).astype(jnp.int32),
                item_ids.reshape(NW, bpw).astype(jnp.int32), ptab)

    dense = _make_dense(B, D, H1, H2, H3)
    w1f = jnp.concatenate(
        [jnp.zeros((2 * D, H1), jnp.float32), W1], axis=0)
    out = dense(
        g,
        w1f, b1.reshape(1, H1),
        W2, b2.reshape(1, H2),
        W3, b3.reshape(1, H3),
        Wo[:D], Wo[D:], bo.reshape(1, 1),
    )
    return out.reshape(B, 1)


# speedup vs baseline: 3.3860x; 1.0042x over previous
import functools

import jax
import jax.numpy as jnp
from jax import lax
from jax.experimental import pallas as pl
from jax.experimental.pallas import tpu as pltpu
from jax.experimental.pallas import tpu_sc as plsc

_IDX_CHUNK = 128
_VBLK = 8192


def _detile_body(gu, gi, mu, mi, ident, out):
    x = jnp.concatenate([gu[...], gi[...], mu[...], mi[...]], axis=0)
    out[...] = jax.lax.dot_general(
        x, ident[...], (((0,), (0,)), ((), ())),
        preferred_element_type=jnp.float32)


@functools.lru_cache(maxsize=None)
def _make_detile(V, D):
    nblk = (V + _VBLK - 1) // _VBLK
    Vp = nblk * _VBLK

    in_spec = pl.BlockSpec((D, _VBLK), lambda i: (0, i))
    return Vp, pl.pallas_call(
        _detile_body,
        grid=(nblk,),
        in_specs=[in_spec] * 4 + [
            pl.BlockSpec((4 * D, 4 * D), lambda i: (0, 0))],
        out_specs=pl.BlockSpec((_VBLK, 4 * D), lambda i: (i, 0)),
        out_shape=jax.ShapeDtypeStruct((Vp, 4 * D), jnp.float32),
        compiler_params=pltpu.CompilerParams(
            fuse_transposed_lhs_in_matmul=True),
    )


@functools.lru_cache(maxsize=None)
def _make_gather4(B, D):
    info = plsc.get_sparse_core_info()
    NC, NS = info.num_cores, info.num_subcores
    NW = NC * NS
    assert B % (NW * _IDX_CHUNK) == 0
    bpw = B // NW
    nchunks = bpw // _IDX_CHUNK

    mesh = plsc.VectorSubcoreMesh(core_axis_name="c", subcore_axis_name="s")

    L = 16

    @functools.partial(
        pl.kernel,
        out_type=jax.ShapeDtypeStruct((B, 4 * D), jnp.float32),
        mesh=mesh,
        scratch_types=[
            pltpu.VMEM((bpw,), jnp.int32),
            pltpu.VMEM((bpw,), jnp.int32),
            pltpu.VMEM((nchunks, _IDX_CHUNK), jnp.int32),
            pltpu.VMEM((nchunks, _IDX_CHUNK), jnp.int32),
            pltpu.VMEM((nchunks, _IDX_CHUNK), jnp.int32),
            pltpu.VMEM((nchunks, _IDX_CHUNK), jnp.int32),
            pltpu.VMEM((bpw, D), jnp.float32),
            pltpu.VMEM((bpw, D), jnp.float32),
            pltpu.VMEM((bpw, D), jnp.float32),
            pltpu.VMEM((bpw, D), jnp.float32),
            pltpu.SemaphoreType.DMA,
            pltpu.SemaphoreType.DMA,
            pltpu.SemaphoreType.DMA,
            pltpu.SemaphoreType.DMA,
            pltpu.SemaphoreType.DMA,
        ],
        compiler_params=pltpu.CompilerParams(use_tc_tiling_on_sc=False),
    )
    def gather4(uids, iids, ptab, out_g,
                u_raw, i_raw, v_gu, v_gi, v_mu, v_mi,
                b_gu, b_gi, b_mu, b_mi,
                s_gu, s_gi, s_mu, s_mi, s_wr):
        wid = lax.axis_index("s") * NC + lax.axis_index("c")
        base = wid * bpw
        pltpu.sync_copy(uids.at[wid], u_raw)
        pltpu.sync_copy(iids.at[wid], i_raw)
        for j in range(bpw // L):
            r, c = j // (_IDX_CHUNK // L), j % (_IDX_CHUNK // L)
            sl = pl.ds(c * L, L)
            mu4 = u_raw[pl.ds(j * L, L)] * 4
            mi4 = i_raw[pl.ds(j * L, L)] * 4
            v_gu[r, sl] = mu4
            v_gi[r, sl] = mi4 + 1
            v_mu[r, sl] = mu4 + 2
            v_mi[r, sl] = mi4 + 3
        copies = []
        for j in range(nchunks):
            rows = pl.ds(j * _IDX_CHUNK, _IDX_CHUNK)
            copies.append(pltpu.async_copy(ptab.at[v_gu.at[j]], b_gu.at[rows], s_gu))
            copies.append(pltpu.async_copy(ptab.at[v_gi.at[j]], b_gi.at[rows], s_gi))
            copies.append(pltpu.async_copy(ptab.at[v_mu.at[j]], b_mu.at[rows], s_mu))
            copies.append(pltpu.async_copy(ptab.at[v_mi.at[j]], b_mi.at[rows], s_mi))
        out_rows = pl.ds(base, bpw)
        writes = []
        for t, (buf, sem) in enumerate(
                [(b_gu, s_gu), (b_gi, s_gi), (b_mu, s_mu), (b_mi, s_mi)]):
            for c in copies[t::4]:
                c.wait()
            writes.append(pltpu.async_copy(
                buf, out_g.at[out_rows, pl.ds(t * D, D)], s_wr))
        for w in writes:
            w.wait()

    return NW, nchunks, gather4


def _dense_body(g_ref, w1f, b1, w2, b2, w3, b3, wog, woh, bo, out):
    x = g_ref[...]
    D = 32
    BLK = x.shape[0]
    L = 4 * D
    gu = x[:, 0 * D:1 * D]
    gi = x[:, 1 * D:2 * D]
    h = x @ w1f[...] + b1[...]
    h = jnp.maximum(h, 0.0)
    h = jnp.maximum(h @ w2[...] + b2[...], 0.0)
    h = jnp.maximum(h @ w3[...] + b3[...], 0.0)
    g = gu * gi
    wog_b = jnp.broadcast_to(wog[...], (D, L))
    woh_b = jnp.broadcast_to(woh[...], (H := h.shape[1], L))
    logit_b = g @ wog_b + h @ woh_b
    rows = jax.lax.broadcasted_iota(jnp.int32, (BLK, L), 0)
    cols = jax.lax.broadcasted_iota(jnp.int32, (BLK, L), 1)
    z = jnp.where(rows % L == cols, logit_b, 0.0)
    ngrp = BLK // L
    sel_rows = jax.lax.broadcasted_iota(jnp.int32, (BLK, ngrp), 0)
    sel_cols = jax.lax.broadcasted_iota(jnp.int32, (BLK, ngrp), 1)
    sel = jnp.where(sel_rows // L == sel_cols, 1.0, 0.0)
    y = jax.lax.dot_general(sel, z, (((0,), (0,)), ((), ())),
                            preferred_element_type=jnp.float32)
    y = y + bo[0, 0]
    out[...] = (1.0 / (1.0 + jnp.exp(-y))).reshape(-1)


@functools.lru_cache(maxsize=None)
def _make_dense(B, D, H1, H2, H3):
    BLK = 4096
    assert B % BLK == 0
    grid = (B // BLK,)

    def wspec(shape):
        return pl.BlockSpec(shape, lambda i: (0,) * len(shape))

    return pl.pallas_call(
        _dense_body,
        grid=grid,
        in_specs=[
            pl.BlockSpec((BLK, 4 * D), lambda i: (i, 0)),
            wspec((4 * D, H1)), wspec((1, H1)),
            wspec((H1, H2)), wspec((1, H2)),
            wspec((H2, H3)), wspec((1, H3)),
            wspec((D, 1)), wspec((H3, 1)), wspec((1, 1)),
        ],
        out_specs=pl.BlockSpec((BLK,), lambda i: (i,)),
        out_shape=jax.ShapeDtypeStruct((B,), jnp.float32),
    )


def kernel(user_ids, item_ids, gmf_user_emb, gmf_item_emb, mlp_user_emb,
           mlp_item_emb, W1, b1, W2, b2, W3, b3, Wo, bo):
    B = user_ids.shape[0]
    V, D = gmf_user_emb.shape
    H1, H2, H3 = W1.shape[1], W2.shape[1], W3.shape[1]

    Vp, detile = _make_detile(V, D)
    ident = jnp.eye(4 * D, dtype=jnp.float32)
    ptab = detile(gmf_user_emb.T, gmf_item_emb.T, mlp_user_emb.T,
                  mlp_item_emb.T, ident)
    ptab = ptab.reshape(Vp * 4, D)

    NW, nchunks, gather4 = _make_gather4(B, D)
    bpw = B // NW
    g = gather4(user_ids.reshape(NW, bpw).astype(jnp.int32),
                item_ids.reshape(NW, bpw).astype(jnp.int32), ptab)

    dense = _make_dense(B, D, H1, H2, H3)
    w1f = jnp.concatenate(
        [jnp.zeros((2 * D, H1), jnp.float32), W1], axis=0)
    out = dense(
        g,
        w1f, b1.reshape(1, H1),
        W2, b2.reshape(1, H2),
        W3, b3.reshape(1, H3),
        Wo[:D], Wo[D:], bo.reshape(1, 1),
    )
    return out.reshape(B, 1)
